# Initial kernel scaffold; baseline (speedup 1.0000x reference)
import jax
import jax.numpy as jnp
from jax.experimental import pallas as pl


def kernel(E_one_hot, t):
    raise NotImplementedError("write your pallas kernel here")


# speedup vs baseline: 69.0232x; 69.0232x over previous
import numpy as np
import jax
import jax.numpy as jnp
from jax.experimental import pallas as pl
from jax.experimental.pallas import tpu as pltpu

_T_E = 1000
_N = 4096
_TINY = np.float32(np.finfo(np.float32).tiny)


def _alpha_bars_table(T=_T_E, s=0.008):
    num_steps = T + 2
    t = np.linspace(0, num_steps, num_steps)
    ab = np.cos(0.5 * np.pi * (t / num_steps + s) / (1 + s)) ** 2
    ab = ab / ab[0]
    alphas = ab[1:] / ab[:-1]
    betas = 1 - alphas
    alphas = 1 - np.clip(betas, 0.0, 0.9999)
    log_ab = np.cumsum(np.log(alphas))
    return np.exp(log_ab).astype(np.float32)


_ALPHA_BARS = jnp.asarray(_alpha_bars_table())
_I_E = jnp.eye(2, dtype=jnp.float32)
_M_E = jnp.broadcast_to(jnp.array([0.9, 0.1], jnp.float32)[None, :], (2, 2))


def _threefry_bits(e):
    ks0 = np.uint32(0)
    ks1 = np.uint32(1)
    ks2 = np.uint32(0x1BD11BDB)
    x0 = jnp.full(e.shape, ks0, jnp.uint32)
    x1 = e + ks1
    rots = ((13, 15, 26, 6), (17, 29, 16, 24))
    inj = ((ks1, ks2), (ks2, ks0), (ks0, ks1), (ks1, ks2), (ks2, ks0))
    for g in range(5):
        for r in rots[g % 2]:
            x0 = x0 + x1
            x1 = (x1 << np.uint32(r)) | (x1 >> np.uint32(32 - r))
            x1 = x1 ^ x0
        a, b = inj[g]
        x0 = x0 + a
        x1 = x1 + b + np.uint32(g + 1)
    return x0 ^ x1


def _gumbel(bits):
    f = jax.lax.bitcast_convert_type(
        (bits >> np.uint32(9)) | np.uint32(0x3F800000), jnp.float32)
    f = f - np.float32(1.0)
    u = jnp.maximum(_TINY, f + _TINY)
    return -jnp.log(-jnp.log(u))


def _body(scal_ref, x_ref, prob_ref, et_ref, *, bm, bn, n):
    bi = pl.program_id(0)
    bj = pl.program_id(1)
    q00 = scal_ref[0]
    q01 = scal_ref[1]
    q10 = scal_ref[2]
    q11 = scal_ref[3]
    l00 = scal_ref[4]
    l01 = scal_ref[5]
    l10 = scal_ref[6]
    l11 = scal_ref[7]

    x = x_ref[:, :]
    par = jax.lax.broadcasted_iota(jnp.int32, (bm, 2 * bn), 1) & 1
    q_on = jnp.where(par == 0, q00, q11)
    q_off = jnp.where(par == 0, q10, q01)
    prob_ref[:, :] = jnp.where(x > 0.5, q_on, q_off)

    kf = jax.lax.broadcasted_iota(jnp.int32, (2 * bn, bn), 0)
    jf = jax.lax.broadcasted_iota(jnp.int32, (2 * bn, bn), 1)
    sel = (kf == 2 * jf + 1).astype(jnp.float32)
    bit = jnp.dot(x, sel, preferred_element_type=jnp.float32)

    ii = bi * bm + jax.lax.broadcasted_iota(jnp.int32, (bm, bn), 0)
    jj = bj * bn + jax.lax.broadcasted_iota(jnp.int32, (bm, bn), 1)
    mn = jnp.minimum(ii, jj)
    mx = jnp.maximum(ii, jj)
    e0 = ((mn * n + mx) * 2).astype(jnp.uint32)
    g0 = _gumbel(_threefry_bits(e0))
    g1 = _gumbel(_threefry_bits(e0 + np.uint32(1)))
    l0 = jnp.where(bit > 0.5, l10, l00)
    l1 = jnp.where(bit > 0.5, l11, l01)
    et_ref[:, :] = (l1 + g1 > l0 + g0).astype(jnp.int32)


def _run(x_flat, scal, n, bm, bn, interpret=False):
    import functools
    body = functools.partial(_body, bm=bm, bn=bn, n=n)
    grid = (n // bm, n // bn)
    return pl.pallas_call(
        body,
        grid=grid,
        in_specs=[
            pl.BlockSpec(memory_space=pltpu.SMEM),
            pl.BlockSpec((bm, 2 * bn), lambda i, j: (i, j)),
        ],
        out_specs=[
            pl.BlockSpec((bm, 2 * bn), lambda i, j: (i, j)),
            pl.BlockSpec((bm, bn), lambda i, j: (i, j)),
        ],
        out_shape=[
            jax.ShapeDtypeStruct((n, 2 * n), jnp.float32),
            jax.ShapeDtypeStruct((n, n), jnp.int32),
        ],
        interpret=interpret,
    )(scal, x_flat)


def kernel(E_one_hot, t):
    alpha_bar_t = _ALPHA_BARS[t]
    Q = alpha_bar_t * _I_E + (1.0 - alpha_bar_t) * _M_E
    qu = jax.lax.bitcast_convert_type(Q, jnp.uint32)
    qu = (qu + np.uint32(0x7FFF) + ((qu >> np.uint32(16)) & np.uint32(1)))
    qu = qu & np.uint32(0xFFFF0000)
    Qb = jax.lax.bitcast_convert_type(qu, jnp.float32)
    lq = jnp.log(jnp.clip(Qb, 1e-30, None))
    scal = jnp.concatenate([Qb.reshape(-1), lq.reshape(-1)])
    x_flat = E_one_hot.reshape(_N, 2 * _N)
    prob_flat, E_t = _run(x_flat, scal, _N, 256, 256)
    prob_E = prob_flat.reshape(_N, _N, 2)
    t_float_E = t.astype(jnp.float32) / float(_T_E)
    return (t_float_E, E_t, prob_E)


# speedup vs baseline: 76.6472x; 1.1105x over previous
import numpy as np
import jax
import jax.numpy as jnp
from jax.experimental import pallas as pl
from jax.experimental.pallas import tpu as pltpu

_T_E = 1000
_N = 4096
_TINY = np.float32(np.finfo(np.float32).tiny)


def _alpha_bars_table(T=_T_E, s=0.008):
    num_steps = T + 2
    t = np.linspace(0, num_steps, num_steps)
    ab = np.cos(0.5 * np.pi * (t / num_steps + s) / (1 + s)) ** 2
    ab = ab / ab[0]
    alphas = ab[1:] / ab[:-1]
    betas = 1 - alphas
    alphas = 1 - np.clip(betas, 0.0, 0.9999)
    log_ab = np.cumsum(np.log(alphas))
    return np.exp(log_ab).astype(np.float32)


_ALPHA_BARS = jnp.asarray(_alpha_bars_table())
_I_E = jnp.eye(2, dtype=jnp.float32)
_M_E = jnp.broadcast_to(jnp.array([0.9, 0.1], jnp.float32)[None, :], (2, 2))


def _threefry_bits(e):
    ks0 = np.uint32(0)
    ks1 = np.uint32(1)
    ks2 = np.uint32(0x1BD11BDB)
    x0 = jnp.full(e.shape, ks0, jnp.uint32)
    x1 = e + ks1
    rots = ((13, 15, 26, 6), (17, 29, 16, 24))
    inj = ((ks1, ks2), (ks2, ks0), (ks0, ks1), (ks1, ks2), (ks2, ks0))
    for g in range(5):
        for r in rots[g % 2]:
            x0 = x0 + x1
            x1 = (x1 << np.uint32(r)) | (x1 >> np.uint32(32 - r))
            x1 = x1 ^ x0
        a, b = inj[g]
        x0 = x0 + a
        x1 = x1 + b + np.uint32(g + 1)
    return x0 ^ x1


def _gumbel(bits):
    f = jax.lax.bitcast_convert_type(
        (bits >> np.uint32(9)) | np.uint32(0x3F800000), jnp.float32)
    f = f - np.float32(1.0)
    u = jnp.maximum(_TINY, f + _TINY)
    return -jnp.log(-jnp.log(u))


def _body(scal_ref, x_ref, prob_ref, et_ref, *, bm, bn, n):
    bi = pl.program_id(0)
    bj = pl.program_id(1)
    q00 = scal_ref[0]
    q01 = scal_ref[1]
    q10 = scal_ref[2]
    q11 = scal_ref[3]
    l00 = scal_ref[4]
    l01 = scal_ref[5]
    l10 = scal_ref[6]
    l11 = scal_ref[7]

    x = x_ref[:, :]
    par = jax.lax.broadcasted_iota(jnp.int32, (bm, 2 * bn), 1) & 1
    q_on = jnp.where(par == 0, q00, q11)
    q_off = jnp.where(par == 0, q10, q01)
    prob_ref[:, :] = jnp.where(x > 0.5, q_on, q_off)

    @pl.when(bi <= bj)
    def _sample():
        kf = jax.lax.broadcasted_iota(jnp.int32, (2 * bn, bn), 0)
        jf = jax.lax.broadcasted_iota(jnp.int32, (2 * bn, bn), 1)
        sel = (kf == 2 * jf + 1).astype(jnp.float32)
        bit = jnp.dot(x, sel, preferred_element_type=jnp.float32)

        ii = bi * bm + jax.lax.broadcasted_iota(jnp.int32, (bm, bn), 0)
        jj = bj * bn + jax.lax.broadcasted_iota(jnp.int32, (bm, bn), 1)
        mn = jnp.minimum(ii, jj)
        mx = jnp.maximum(ii, jj)
        e0 = ((mn * n + mx) * 2).astype(jnp.uint32)
        g0 = _gumbel(_threefry_bits(e0))
        g1 = _gumbel(_threefry_bits(e0 + np.uint32(1)))
        l0 = jnp.where(bit > 0.5, l10, l00)
        l1 = jnp.where(bit > 0.5, l11, l01)
        et_ref[:, :] = (l1 + g1 > l0 + g0).astype(jnp.int32)


def _mirror_body(et_in_ref, et_out_ref):
    bi = pl.program_id(0)
    bj = pl.program_id(1)
    blk = et_in_ref[:, :]
    bt = blk.shape[0]

    @pl.when(bi > bj)
    def _lower():
        et_out_ref[:, :] = blk.T

    @pl.when(bi < bj)
    def _upper():
        et_out_ref[:, :] = blk

    @pl.when(bi == bj)
    def _diag():
        ii = jax.lax.broadcasted_iota(jnp.int32, (bt, bt), 0)
        jj = jax.lax.broadcasted_iota(jnp.int32, (bt, bt), 1)
        et_out_ref[:, :] = jnp.where(ii <= jj, blk, blk.T)


def _run(x_flat, scal, n, bm, bn, bt, interpret=False):
    import functools
    body = functools.partial(_body, bm=bm, bn=bn, n=n)
    grid = (n // bm, n // bn)
    prob_flat, et_part = pl.pallas_call(
        body,
        grid=grid,
        in_specs=[
            pl.BlockSpec(memory_space=pltpu.SMEM),
            pl.BlockSpec((bm, 2 * bn), lambda i, j: (i, j)),
        ],
        out_specs=[
            pl.BlockSpec((bm, 2 * bn), lambda i, j: (i, j)),
            pl.BlockSpec((bm, bn), lambda i, j: (i, jnp.maximum(i, j))),
        ],
        out_shape=[
            jax.ShapeDtypeStruct((n, 2 * n), jnp.float32),
            jax.ShapeDtypeStruct((n, n), jnp.int32),
        ],
        interpret=interpret,
    )(scal, x_flat)
    gt = n // bt
    et = pl.pallas_call(
        _mirror_body,
        grid=(gt, gt),
        in_specs=[pl.BlockSpec(
            (bt, bt), lambda i, j: (jnp.minimum(i, j), jnp.maximum(i, j)))],
        out_specs=pl.BlockSpec((bt, bt), lambda i, j: (i, j)),
        out_shape=jax.ShapeDtypeStruct((n, n), jnp.int32),
        interpret=interpret,
    )(et_part)
    return prob_flat, et


def kernel(E_one_hot, t):
    alpha_bar_t = _ALPHA_BARS[t]
    Q = alpha_bar_t * _I_E + (1.0 - alpha_bar_t) * _M_E
    qu = jax.lax.bitcast_convert_type(Q, jnp.uint32)
    qu = (qu + np.uint32(0x7FFF) + ((qu >> np.uint32(16)) & np.uint32(1)))
    qu = qu & np.uint32(0xFFFF0000)
    Qb = jax.lax.bitcast_convert_type(qu, jnp.float32)
    lq = jnp.log(jnp.clip(Qb, 1e-30, None))
    scal = jnp.concatenate([Qb.reshape(-1), lq.reshape(-1)])
    x_flat = E_one_hot.reshape(_N, 2 * _N)
    prob_flat, E_t = _run(x_flat, scal, _N, 256, 256, 1024)
    prob_E = prob_flat.reshape(_N, _N, 2)
    t_float_E = t.astype(jnp.float32) / float(_T_E)
    return (t_float_E, E_t, prob_E)


# speedup vs baseline: 147.8427x; 1.9289x over previous
import numpy as np
import jax
import jax.numpy as jnp
from jax.experimental import pallas as pl
from jax.experimental.pallas import tpu as pltpu

_T_E = 1000
_N = 4096
_TINY = np.float32(np.finfo(np.float32).tiny)


def _alpha_bars_table(T=_T_E, s=0.008):
    num_steps = T + 2
    t = np.linspace(0, num_steps, num_steps)
    ab = np.cos(0.5 * np.pi * (t / num_steps + s) / (1 + s)) ** 2
    ab = ab / ab[0]
    alphas = ab[1:] / ab[:-1]
    betas = 1 - alphas
    alphas = 1 - np.clip(betas, 0.0, 0.9999)
    log_ab = np.cumsum(np.log(alphas))
    return np.exp(log_ab).astype(np.float32)


_ALPHA_BARS = jnp.asarray(_alpha_bars_table())
_I_E = jnp.eye(2, dtype=jnp.float32)
_M_E = jnp.broadcast_to(jnp.array([0.9, 0.1], jnp.float32)[None, :], (2, 2))


def _threefry_bits(e):
    ks0 = np.uint32(0)
    ks1 = np.uint32(1)
    ks2 = np.uint32(0x1BD11BDB)
    x0 = jnp.full(e.shape, ks0, jnp.uint32)
    x1 = e + ks1
    rots = ((13, 15, 26, 6), (17, 29, 16, 24))
    inj = ((ks1, ks2), (ks2, ks0), (ks0, ks1), (ks1, ks2), (ks2, ks0))
    for g in range(5):
        for r in rots[g % 2]:
            x0 = x0 + x1
            x1 = (x1 << np.uint32(r)) | (x1 >> np.uint32(32 - r))
            x1 = x1 ^ x0
        a, b = inj[g]
        x0 = x0 + a
        x1 = x1 + b + np.uint32(g + 1)
    return x0 ^ x1


def _gumbel(bits):
    f = jax.lax.bitcast_convert_type(
        (bits >> np.uint32(9)) | np.uint32(0x3F800000), jnp.float32)
    f = f - np.float32(1.0)
    u = jnp.maximum(_TINY, f + _TINY)
    return -jnp.log(-jnp.log(u))


def _body(scal_ref, x_ref, prob_ref, et_ref, *, bm, bn, n):
    bi = pl.program_id(0)
    bk = pl.program_id(1)
    bnt = bn // 128
    q00 = scal_ref[0]
    q01 = scal_ref[1]
    q10 = scal_ref[2]
    q11 = scal_ref[3]
    l00 = scal_ref[4]
    l01 = scal_ref[5]
    l10 = scal_ref[6]
    l11 = scal_ref[7]

    do_sample = bi * bm < (bk + 1) * bn
    for tt in range(bnt):
        bit = x_ref[:, 2 * tt + 1, :]
        prob_ref[:, 2 * tt, :] = jnp.where(bit > 0.5, q10, q00)
        prob_ref[:, 2 * tt + 1, :] = jnp.where(bit > 0.5, q11, q01)

        @pl.when(do_sample)
        def _sample():
            ii = bi * bm + jax.lax.broadcasted_iota(jnp.int32, (bm, 128), 0)
            jj = (bk * bn + tt * 128
                  + jax.lax.broadcasted_iota(jnp.int32, (bm, 128), 1))
            mn = jnp.minimum(ii, jj)
            mx = jnp.maximum(ii, jj)
            e0 = ((mn * n + mx) * 2).astype(jnp.uint32)
            g0 = _gumbel(_threefry_bits(e0))
            g1 = _gumbel(_threefry_bits(e0 + np.uint32(1)))
            l0 = jnp.where(bit > 0.5, l10, l00)
            l1 = jnp.where(bit > 0.5, l11, l01)
            et_ref[:, tt * 128:(tt + 1) * 128] = (
                l1 + g1 > l0 + g0).astype(jnp.int32)


def _mirror_body(et_in_ref, et_out_ref):
    bi = pl.program_id(0)
    bj = pl.program_id(1)
    blk = et_in_ref[:, :]
    bt = blk.shape[0]

    @pl.when(bi > bj)
    def _lower():
        et_out_ref[:, :] = blk.T

    @pl.when(bi < bj)
    def _upper():
        et_out_ref[:, :] = blk

    @pl.when(bi == bj)
    def _diag():
        ii = jax.lax.broadcasted_iota(jnp.int32, (bt, bt), 0)
        jj = jax.lax.broadcasted_iota(jnp.int32, (bt, bt), 1)
        et_out_ref[:, :] = jnp.where(ii <= jj, blk, blk.T)


def _run(x_z, scal, n, bm, bn, bt, interpret=False):
    import functools
    body = functools.partial(_body, bm=bm, bn=bn, n=n)
    gm, gk = n // bm, n // bn
    bnt = bn // 128

    def _et_map(i, k):
        return (i, jnp.minimum(jnp.maximum(k, (i * bm) // bn), gk - 1))

    prob_z, et_part = pl.pallas_call(
        body,
        grid=(gm, gk),
        in_specs=[
            pl.BlockSpec(memory_space=pltpu.SMEM),
            pl.BlockSpec((bm, 2 * bnt, 128), lambda i, k: (i, k, 0)),
        ],
        out_specs=[
            pl.BlockSpec((bm, 2 * bnt, 128), lambda i, k: (i, k, 0)),
            pl.BlockSpec((bm, bn), _et_map),
        ],
        out_shape=[
            jax.ShapeDtypeStruct((n, 2 * (n // 128), 128), jnp.float32),
            jax.ShapeDtypeStruct((n, n), jnp.int32),
        ],
        interpret=interpret,
    )(scal, x_z)
    gt = n // bt
    et = pl.pallas_call(
        _mirror_body,
        grid=(gt, gt),
        in_specs=[pl.BlockSpec(
            (bt, bt), lambda i, j: (jnp.minimum(i, j), jnp.maximum(i, j)))],
        out_specs=pl.BlockSpec((bt, bt), lambda i, j: (i, j)),
        out_shape=jax.ShapeDtypeStruct((n, n), jnp.int32),
        interpret=interpret,
    )(et_part)
    return prob_z, et


def kernel(E_one_hot, t):
    alpha_bar_t = _ALPHA_BARS[t]
    Q = alpha_bar_t * _I_E + (1.0 - alpha_bar_t) * _M_E
    qu = jax.lax.bitcast_convert_type(Q, jnp.uint32)
    qu = (qu + np.uint32(0x7FFF) + ((qu >> np.uint32(16)) & np.uint32(1)))
    qu = qu & np.uint32(0xFFFF0000)
    Qb = jax.lax.bitcast_convert_type(qu, jnp.float32)
    lq = jnp.log(jnp.clip(Qb, 1e-30, None))
    scal = jnp.concatenate([Qb.reshape(-1), lq.reshape(-1)])
    nt = _N // 128
    x_z = (E_one_hot.reshape(_N, nt, 128, 2)
           .transpose(0, 1, 3, 2).reshape(_N, 2 * nt, 128))
    prob_z, E_t = _run(x_z, scal, _N, 256, 512, 1024)
    prob_E = (prob_z.reshape(_N, nt, 2, 128)
              .transpose(0, 1, 3, 2).reshape(_N, _N, 2))
    t_float_E = t.astype(jnp.float32) / float(_T_E)
    return (t_float_E, E_t, prob_E)


# speedup vs baseline: 154.1837x; 1.0429x over previous
import numpy as np
import jax
import jax.numpy as jnp
from jax.experimental import pallas as pl
from jax.experimental.pallas import tpu as pltpu

_T_E = 1000
_N = 4096
_TINY = np.float32(np.finfo(np.float32).tiny)


def _alpha_bars_table(T=_T_E, s=0.008):
    num_steps = T + 2
    t = np.linspace(0, num_steps, num_steps)
    ab = np.cos(0.5 * np.pi * (t / num_steps + s) / (1 + s)) ** 2
    ab = ab / ab[0]
    alphas = ab[1:] / ab[:-1]
    betas = 1 - alphas
    alphas = 1 - np.clip(betas, 0.0, 0.9999)
    log_ab = np.cumsum(np.log(alphas))
    return np.exp(log_ab).astype(np.float32)


_ALPHA_BARS = jnp.asarray(_alpha_bars_table())
_I_E = jnp.eye(2, dtype=jnp.float32)
_M_E = jnp.broadcast_to(jnp.array([0.9, 0.1], jnp.float32)[None, :], (2, 2))


def _threefry_bits(e):
    ks0 = np.uint32(0)
    ks1 = np.uint32(1)
    ks2 = np.uint32(0x1BD11BDB)
    x0 = jnp.full(e.shape, ks0, jnp.uint32)
    x1 = e + ks1
    rots = ((13, 15, 26, 6), (17, 29, 16, 24))
    inj = ((ks1, ks2), (ks2, ks0), (ks0, ks1), (ks1, ks2), (ks2, ks0))
    for g in range(5):
        for r in rots[g % 2]:
            x0 = x0 + x1
            x1 = (x1 << np.uint32(r)) | (x1 >> np.uint32(32 - r))
            x1 = x1 ^ x0
        a, b = inj[g]
        x0 = x0 + a
        x1 = x1 + b + np.uint32(g + 1)
    return x0 ^ x1


def _gumbel(bits):
    f = jax.lax.bitcast_convert_type(
        (bits >> np.uint32(9)) | np.uint32(0x3F800000), jnp.float32)
    f = f - np.float32(1.0)
    u = jnp.maximum(_TINY, f + _TINY)
    return -jnp.log(-jnp.log(u))


def _body(scal_ref, x_ref, prob_ref, et_ref, *, bm, bn, n):
    bi = pl.program_id(0)
    bk = pl.program_id(1)
    bnt = bn // 128
    q00 = scal_ref[0]
    q01 = scal_ref[1]
    q10 = scal_ref[2]
    q11 = scal_ref[3]
    l00 = scal_ref[4]
    l01 = scal_ref[5]
    l10 = scal_ref[6]
    l11 = scal_ref[7]

    do_sample = bi * bm < (bk + 1) * bn
    for tt in range(bnt):
        bit = x_ref[:, 2 * tt + 1, :]
        prob_ref[:, 2 * tt, :] = jnp.where(bit > 0.5, q10, q00)
        prob_ref[:, 2 * tt + 1, :] = jnp.where(bit > 0.5, q11, q01)

        @pl.when(do_sample)
        def _sample():
            ii = bi * bm + jax.lax.broadcasted_iota(jnp.int32, (bm, 128), 0)
            jj = (bk * bn + tt * 128
                  + jax.lax.broadcasted_iota(jnp.int32, (bm, 128), 1))
            mn = jnp.minimum(ii, jj)
            mx = jnp.maximum(ii, jj)
            e0 = ((mn * n + mx) * 2).astype(jnp.uint32)
            g0 = _gumbel(_threefry_bits(e0))
            g1 = _gumbel(_threefry_bits(e0 + np.uint32(1)))
            l0 = jnp.where(bit > 0.5, l10, l00)
            l1 = jnp.where(bit > 0.5, l11, l01)
            et_ref[:, tt * 128:(tt + 1) * 128] = (
                l1 + g1 > l0 + g0).astype(jnp.int32)


def _mirror_body(et_in_ref, et_out_ref):
    bi = pl.program_id(0)
    bj = pl.program_id(1)
    blk = et_in_ref[:, :]
    bt = blk.shape[0]

    @pl.when(bi > bj)
    def _lower():
        et_out_ref[:, :] = blk.T

    @pl.when(bi < bj)
    def _upper():
        et_out_ref[:, :] = blk

    @pl.when(bi == bj)
    def _diag():
        ii = jax.lax.broadcasted_iota(jnp.int32, (bt, bt), 0)
        jj = jax.lax.broadcasted_iota(jnp.int32, (bt, bt), 1)
        et_out_ref[:, :] = jnp.where(ii <= jj, blk, blk.T)


def _run(x_z, scal, n, bm, bn, bt, interpret=False):
    import functools
    body = functools.partial(_body, bm=bm, bn=bn, n=n)
    gm, gk = n // bm, n // bn
    bnt = bn // 128

    def _et_map(i, k):
        return (i, jnp.minimum(jnp.maximum(k, (i * bm) // bn), gk - 1))

    prob_z, et_part = pl.pallas_call(
        body,
        grid=(gm, gk),
        in_specs=[
            pl.BlockSpec(memory_space=pltpu.SMEM),
            pl.BlockSpec((bm, 2 * bnt, 128), lambda i, k: (i, k, 0)),
        ],
        out_specs=[
            pl.BlockSpec((bm, 2 * bnt, 128), lambda i, k: (i, k, 0)),
            pl.BlockSpec((bm, bn), _et_map),
        ],
        out_shape=[
            jax.ShapeDtypeStruct((n, 2 * (n // 128), 128), jnp.float32),
            jax.ShapeDtypeStruct((n, n), jnp.int32),
        ],
        interpret=interpret,
    )(scal, x_z)
    gt = n // bt
    et = pl.pallas_call(
        _mirror_body,
        grid=(gt, gt),
        in_specs=[pl.BlockSpec(
            (bt, bt), lambda i, j: (jnp.minimum(i, j), jnp.maximum(i, j)))],
        out_specs=pl.BlockSpec((bt, bt), lambda i, j: (i, j)),
        out_shape=jax.ShapeDtypeStruct((n, n), jnp.int32),
        interpret=interpret,
    )(et_part)
    return prob_z, et


def kernel(E_one_hot, t):
    alpha_bar_t = _ALPHA_BARS[t]
    Q = alpha_bar_t * _I_E + (1.0 - alpha_bar_t) * _M_E
    qu = jax.lax.bitcast_convert_type(Q, jnp.uint32)
    qu = (qu + np.uint32(0x7FFF) + ((qu >> np.uint32(16)) & np.uint32(1)))
    qu = qu & np.uint32(0xFFFF0000)
    Qb = jax.lax.bitcast_convert_type(qu, jnp.float32)
    lq = jnp.log(jnp.clip(Qb, 1e-30, None))
    scal = jnp.concatenate([Qb.reshape(-1), lq.reshape(-1)])
    nt = _N // 128
    x_z = (E_one_hot.reshape(_N, nt, 128, 2)
           .transpose(0, 1, 3, 2).reshape(_N, 2 * nt, 128))
    prob_z, E_t = _run(x_z, scal, _N, 512, 512, 1024)
    prob_E = (prob_z.reshape(_N, nt, 2, 128)
              .transpose(0, 1, 3, 2).reshape(_N, _N, 2))
    t_float_E = t.astype(jnp.float32) / float(_T_E)
    return (t_float_E, E_t, prob_E)
